# baseline (speedup 1.0000x reference)
import functools

import jax
import jax.numpy as jnp
from jax import lax
from jax.experimental import pallas as pl
from jax.experimental.pallas import tpu as pltpu
from jax.experimental.pallas import tpu_sc as plsc

_NUM_FIELDS = 26
_VOCAB = 100000
_EMB = 32
_BATCH = 16384
_OUT_D = _NUM_FIELDS * _EMB
_NW = 32
_BPW = _BATCH // _NW
_CB = 64
_NCHK = _BPW // _CB

_mesh = plsc.VectorSubcoreMesh(core_axis_name="c", subcore_axis_name="s")


@functools.partial(
    pl.kernel,
    mesh=_mesh,
    out_type=jax.ShapeDtypeStruct((_BATCH, _OUT_D), jnp.float32),
    scratch_types=[
        pltpu.VMEM((_NUM_FIELDS, _BPW), jnp.int32),
        pltpu.VMEM((2, _NUM_FIELDS, _CB, _EMB), jnp.float32),
        pltpu.SemaphoreType.DMA,
        pltpu.SemaphoreType.DMA,
        pltpu.SemaphoreType.DMA,
        pltpu.SemaphoreType.DMA,
    ],
    compiler_params=pltpu.CompilerParams(use_tc_tiling_on_sc=False),
)
def _sc_embed(xcatt_hbm, tables_hbm, out_hbm, xvt, stag,
              sem_x, sem_g0, sem_g1, sem_st):
    wid = lax.axis_index("s") * 2 + lax.axis_index("c")
    wb0 = wid * _BPW
    sem_g = [sem_g0, sem_g1]

    xd = [
        pltpu.async_copy(xcatt_hbm.at[i, pl.ds(wb0, _BPW)], xvt.at[i], sem_x)
        for i in range(_NUM_FIELDS)
    ]
    for d in xd:
        d.wait()

    def gathers(c):
        buf = c % 2
        return [
            pltpu.async_copy(
                tables_hbm.at[i].at[xvt.at[i, pl.ds(c * _CB, _CB)]],
                stag.at[buf, i],
                sem_g[buf])
            for i in range(_NUM_FIELDS)
        ]

    def stores(c):
        buf = c % 2
        return [
            pltpu.async_copy(
                stag.at[buf, i],
                out_hbm.at[pl.ds(wb0 + c * _CB, _CB),
                           pl.ds(i * _EMB, _EMB)],
                sem_st)
            for i in range(_NUM_FIELDS)
        ]

    gd = {0: gathers(0)}
    sd = {}
    for c in range(_NCHK):
        if c + 1 < _NCHK:
            if c >= 1:
                for d in sd.pop(c - 1):
                    d.wait()
            gd[c + 1] = gathers(c + 1)
        for d in gd.pop(c):
            d.wait()
        sd[c] = stores(c)
    for d in sd.pop(_NCHK - 2):
        d.wait()
    for d in sd.pop(_NCHK - 1):
        d.wait()


def kernel(x_cat, tables):
    return _sc_embed(x_cat.T, tables)


# speedup vs baseline: 1.4423x; 1.4423x over previous
import functools

import jax
import jax.numpy as jnp
from jax import lax
from jax.experimental import pallas as pl
from jax.experimental.pallas import tpu as pltpu
from jax.experimental.pallas import tpu_sc as plsc

_NUM_FIELDS = 26
_VOCAB = 100000
_EMB = 32
_BATCH = 16384
_NCOMP = _NUM_FIELDS * _EMB
_NW = 32
_CPW = _NCOMP // _NW
_CH = 2048
_NCHK = _BATCH // _CH
_LANES = 16

_mesh = plsc.VectorSubcoreMesh(core_axis_name="c", subcore_axis_name="s")


@functools.partial(
    pl.kernel,
    mesh=_mesh,
    out_type=jax.ShapeDtypeStruct((_NCOMP, _BATCH), jnp.float32),
    scratch_types=[
        pltpu.VMEM((_VOCAB,), jnp.float32),
        pltpu.VMEM((_BATCH,), jnp.int32),
        pltpu.VMEM((2, _CH), jnp.float32),
        pltpu.SemaphoreType.DMA,
        pltpu.SemaphoreType.DMA,
        pltpu.SemaphoreType.DMA,
    ],
    compiler_params=pltpu.CompilerParams(
        use_tc_tiling_on_sc=False, needs_layout_passes=False),
)
def _sc_embed(xt_hbm, tablest_hbm, out_hbm, vec, xrow, ob,
              sem_v, sem_x, sem_st):
    wid = lax.axis_index("s") * 2 + lax.axis_index("c")
    c0 = wid * _CPW

    for j in range(_CPW):
        cc = c0 + j
        fld = cc // _EMB
        comp = lax.rem(cc, _EMB)

        if j == 0:
            pltpu.sync_copy(xt_hbm.at[fld], xrow)
        else:
            prev_fld = (cc - 1) // _EMB

            @pl.when(fld != prev_fld)
            def _():
                pltpu.sync_copy(xt_hbm.at[fld], xrow)

        pltpu.async_copy(tablest_hbm.at[fld, comp], vec, sem_v).wait()

        sd = {}
        for k in range(_NCHK):
            buf = k % 2
            if k >= 2:
                sd.pop(k - 2).wait()

            def gstep(t, carry):
                idx = xrow[pl.ds(k * _CH + t * _LANES, _LANES)]
                ob[buf, pl.ds(t * _LANES, _LANES)] = plsc.load_gather(
                    vec, [idx])
                return carry

            lax.fori_loop(0, _CH // _LANES, gstep, 0, unroll=8)
            sd[k] = pltpu.async_copy(
                ob.at[buf], out_hbm.at[cc, pl.ds(k * _CH, _CH)], sem_st)
        sd.pop(_NCHK - 2).wait()
        sd.pop(_NCHK - 1).wait()


def kernel(x_cat, tables):
    out_t = _sc_embed(x_cat.T, tables.transpose(0, 2, 1))
    return out_t.T
